# baseline (speedup 1.0000x reference)
import functools

import jax
import jax.numpy as jnp
from jax import lax
from jax.experimental import pallas as pl
from jax.experimental.pallas import tpu as pltpu
from jax.experimental.pallas import tpu_sc as plsc

N = 50000
E = 800000
N_PAD = 50176
ROWS_PER_TILE = N_PAD // 16
ZROWS = ROWS_PER_TILE // 16
K = 64
S = 4
E_PAD = 819200
TILE_ROWS = E_PAD // (16 * K)
OUTER = TILE_ROWS // S
EDGES_PER_TILE = E // 16
DEG_CHUNK = 2000

_MESH = plsc.VectorSubcoreMesh(core_axis_name="c", subcore_axis_name="s")
_SC_PARAMS = pltpu.CompilerParams(use_tc_tiling_on_sc=False,
                                  needs_layout_passes=False)


@functools.partial(
    pl.kernel,
    out_type=jax.ShapeDtypeStruct((2 * 16 * N_PAD,), jnp.float32),
    mesh=_MESH,
    scratch_types=[
        pltpu.VMEM((N_PAD,), jnp.float32),
        pltpu.VMEM((DEG_CHUNK,), jnp.int32),
    ],
    compiler_params=_SC_PARAMS,
)
def _degree_kernel(src_ref, dst_ref, out_ref, cnt, idxbuf):
    c = lax.axis_index("c")
    s = lax.axis_index("s")
    z = jnp.zeros((16,), jnp.float32)
    ones = jnp.ones((16,), jnp.float32)

    @pl.loop(0, N_PAD // 16)
    def _zero(i):
        cnt[pl.ds(i * 16, 16)] = z

    def count(idx_ref):
        @pl.loop(0, EDGES_PER_TILE // DEG_CHUNK)
        def _outer(o):
            base = pl.multiple_of(s * EDGES_PER_TILE + o * DEG_CHUNK, 8)
            pltpu.sync_copy(idx_ref.at[pl.ds(base, DEG_CHUNK)], idxbuf)

            @pl.loop(0, DEG_CHUNK // 16)
            def _inner(i):
                idxv = idxbuf[pl.ds(i * 16, 16)]
                plsc.addupdate_scatter(cnt, [idxv], ones)

    @pl.when(c == 0)
    def _():
        count(src_ref)

    @pl.when(c == 1)
    def _():
        count(dst_ref)

    obase = pl.multiple_of((c * 16 + s) * N_PAD, 128)
    pltpu.sync_copy(cnt, out_ref.at[pl.ds(obase, N_PAD)])


def _make_agg(n_chunks):
    @functools.partial(
        pl.kernel,
        out_type=jax.ShapeDtypeStruct((n_chunks, N_PAD, 32), jnp.float32),
        mesh=_MESH,
        scratch_types=[
            pltpu.VMEM_SHARED((N_PAD, 32), jnp.float32),
            pltpu.VMEM((3, S, K), jnp.int32),
            pltpu.VMEM((3, S, K), jnp.int32),
            pltpu.VMEM((2, S, K, 32), jnp.float32),
            pltpu.VMEM((ZROWS, 32), jnp.float32),
            pltpu.SemaphoreType.DMA,
            pltpu.SemaphoreType.DMA,
            pltpu.SemaphoreType.DMA,
        ],
        compiler_params=_SC_PARAMS,
    )
    def _agg(src_ref, dst_ref, *rest):
        h_refs = rest[:n_chunks]
        out_ref = rest[n_chunks]
        acc, srcbuf, dstbuf, msg, zbuf, gsem, ssem, isem = rest[n_chunks + 1:]
        c = lax.axis_index("c")
        s = lax.axis_index("s")
        z = jnp.zeros((16,), jnp.float32)
        zi = jnp.zeros((16,), jnp.int32)

        @pl.loop(0, ZROWS)
        def _zfill(i):
            zbuf[i, pl.ds(0, 16)] = z
            zbuf[i, pl.ds(16, 16)] = z

        for j in range(S):
            @pl.loop(0, K // 16)
            def _dzero(i, j=j):
                dstbuf[0, j, pl.ds(i * 16, 16)] = zi

        def do_chunk(h_ref, chunk):
            rbase = pl.multiple_of(s * ROWS_PER_TILE, 8)

            @pl.loop(0, ROWS_PER_TILE // ZROWS)
            def _za(j):
                pltpu.sync_copy(zbuf, acc.at[pl.ds(rbase + j * ZROWS, ZROWS)])
            plsc.subcore_barrier()

            for j in range(S):
                pltpu.async_copy(zbuf.at[pl.ds(0, K)],
                                 acc.at[dstbuf.at[0, j]], ssem, add=True)
            row0 = s * TILE_ROWS
            pltpu.sync_copy(src_ref.at[pl.ds(row0, S)], srcbuf.at[0])
            pltpu.sync_copy(dst_ref.at[pl.ds(row0, S)], dstbuf.at[0])

            @pl.loop(0, OUTER)
            def _edges(o):
                ib = lax.rem(o, 3)
                nib = lax.rem(o + 1, 3)
                mb = lax.rem(o, 2)

                @pl.when(o > 0)
                def _():
                    pltpu.make_async_copy(
                        src_ref.at[pl.ds(row0, S)], srcbuf.at[0], isem).wait()
                    pltpu.make_async_copy(
                        dst_ref.at[pl.ds(row0, S)], dstbuf.at[0], isem).wait()

                gs = [pltpu.async_copy(h_ref.at[srcbuf.at[ib, j]],
                                       msg.at[mb, j], gsem)
                      for j in range(S)]

                @pl.when(o < OUTER - 1)
                def _():
                    rown = s * TILE_ROWS + (o + 1) * S
                    pltpu.async_copy(src_ref.at[pl.ds(rown, S)],
                                     srcbuf.at[nib], isem)
                    pltpu.async_copy(dst_ref.at[pl.ds(rown, S)],
                                     dstbuf.at[nib], isem)

                for g in gs:
                    g.wait()
                for j in range(S):
                    pltpu.make_async_copy(
                        msg.at[0, j], acc.at[dstbuf.at[0, j]], ssem).wait()
                for j in range(S):
                    pltpu.async_copy(msg.at[mb, j], acc.at[dstbuf.at[ib, j]],
                                     ssem, add=True)

            for j in range(S):
                pltpu.make_async_copy(
                    msg.at[0, j], acc.at[dstbuf.at[0, j]], ssem).wait()

            plsc.subcore_barrier()
            pltpu.sync_copy(
                acc.at[pl.ds(rbase, ROWS_PER_TILE)],
                out_ref.at[chunk, pl.ds(rbase, ROWS_PER_TILE)])

        for phase in range(n_chunks // 2):
            for core in range(2):
                chunk = 2 * phase + core

                @pl.when(c == core)
                def _(h_ref=h_refs[chunk], chunk=chunk):
                    do_chunk(h_ref, chunk)

    return _agg


_agg4 = _make_agg(4)
_agg2 = _make_agg(2)


BLK = 1024
GRID = N_PAD // BLK


def _isqrts(pblk):
    deg = jnp.maximum(jnp.sum(pblk, axis=1), 1.0)
    return lax.rsqrt(deg)


def _stage1_body(feat_ref, p_ref, w1_ref, h1_ref):
    sc = _isqrts(p_ref[...])
    x = feat_ref[...] * sc[0][:, None]
    h1_ref[...] = jnp.dot(x, w1_ref[...], preferred_element_type=jnp.float32)


_stage1 = pl.pallas_call(
    _stage1_body,
    grid=(GRID,),
    in_specs=[
        pl.BlockSpec((BLK, 128), lambda i: (i, 0)),
        pl.BlockSpec((2, 16, BLK), lambda i: (0, 0, i)),
        pl.BlockSpec((128, 128), lambda i: (0, 0)),
    ],
    out_specs=pl.BlockSpec((BLK, 128), lambda i: (i, 0)),
    out_shape=jax.ShapeDtypeStruct((N_PAD, 128), jnp.float32),
)


def _stage2_body(agg_ref, p_ref, b1_ref, w2_ref, h2_ref):
    sc = _isqrts(p_ref[...])
    x1 = jnp.maximum(agg_ref[...] * sc[1][:, None] + b1_ref[...], 0.0)
    h2_ref[...] = jnp.dot(x1 * sc[0][:, None], w2_ref[...],
                          preferred_element_type=jnp.float32)


_stage2 = pl.pallas_call(
    _stage2_body,
    grid=(GRID,),
    in_specs=[
        pl.BlockSpec((BLK, 128), lambda i: (i, 0)),
        pl.BlockSpec((2, 16, BLK), lambda i: (0, 0, i)),
        pl.BlockSpec((1, 128), lambda i: (0, 0)),
        pl.BlockSpec((128, 64), lambda i: (0, 0)),
    ],
    out_specs=pl.BlockSpec((BLK, 64), lambda i: (i, 0)),
    out_shape=jax.ShapeDtypeStruct((N_PAD, 64), jnp.float32),
)


def _stage3_body(agg_ref, p_ref, b2_ref, wo1_ref, bo1_ref, wo2_ref, bo2_ref,
                 hid_ref, log_ref):
    sc = _isqrts(p_ref[...])
    hid = agg_ref[...] * sc[1][:, None] + b2_ref[...]
    hid_ref[...] = hid
    t = jnp.maximum(
        jnp.dot(hid, wo1_ref[...], preferred_element_type=jnp.float32)
        + bo1_ref[...], 0.0)
    log_ref[...] = jnp.dot(t, wo2_ref[...],
                           preferred_element_type=jnp.float32) + bo2_ref[...]


_stage3 = pl.pallas_call(
    _stage3_body,
    grid=(GRID,),
    in_specs=[
        pl.BlockSpec((BLK, 64), lambda i: (i, 0)),
        pl.BlockSpec((2, 16, BLK), lambda i: (0, 0, i)),
        pl.BlockSpec((1, 64), lambda i: (0, 0)),
        pl.BlockSpec((64, 128), lambda i: (0, 0)),
        pl.BlockSpec((1, 128), lambda i: (0, 0)),
        pl.BlockSpec((128, 128), lambda i: (0, 0)),
        pl.BlockSpec((1, 128), lambda i: (0, 0)),
    ],
    out_specs=[
        pl.BlockSpec((BLK, 64), lambda i: (i, 0)),
        pl.BlockSpec((BLK, 128), lambda i: (i, 0)),
    ],
    out_shape=[
        jax.ShapeDtypeStruct((N_PAD, 64), jnp.float32),
        jax.ShapeDtypeStruct((N_PAD, 128), jnp.float32),
    ],
)


def kernel(features, edge_index, W1, b1, W2, b2, Wo1, bo1, Wo2, bo2):
    n = features.shape[0]
    feat = features.reshape(n, -1)
    featp = jnp.pad(feat, ((0, N_PAD - n), (0, 0)))
    src = edge_index[0]
    dst = edge_index[1]
    npad_e = E_PAD - E
    src2d = jnp.concatenate(
        [src, jnp.zeros((npad_e,), jnp.int32)]).reshape(-1, K)
    dst2d = jnp.concatenate(
        [dst, jnp.full((npad_e,), N_PAD - 1, jnp.int32)]).reshape(-1, K)

    partials = _degree_kernel(src, dst).reshape(2, 16, N_PAD)

    h1 = _stage1(featp, partials, W1)
    h1t = h1.reshape(N_PAD, 4, 32).transpose(1, 0, 2)
    agg1t = _agg4(src2d, dst2d, h1t[0], h1t[1], h1t[2], h1t[3])
    agg1 = agg1t.transpose(1, 0, 2).reshape(N_PAD, 128)

    h2 = _stage2(agg1, partials, b1.reshape(1, -1), W2)
    h2t = h2.reshape(N_PAD, 2, 32).transpose(1, 0, 2)
    agg2t = _agg2(src2d, dst2d, h2t[0], h2t[1])
    agg2 = agg2t.transpose(1, 0, 2).reshape(N_PAD, 64)

    wo2p = jnp.pad(Wo2, ((0, 0), (0, 128 - Wo2.shape[1])))
    bo2p = jnp.pad(bo2, (0, 128 - bo2.shape[0])).reshape(1, -1)
    hid_p, log_p = _stage3(agg2, partials, b2.reshape(1, -1), Wo1,
                           bo1.reshape(1, -1), wo2p, bo2p)
    logits = log_p[:n, :2]
    hidden = hid_p[:n]
    return (logits, logits, hidden)


# speedup vs baseline: 1.0368x; 1.0368x over previous
import functools

import jax
import jax.numpy as jnp
from jax import lax
from jax.experimental import pallas as pl
from jax.experimental.pallas import tpu as pltpu
from jax.experimental.pallas import tpu_sc as plsc

N = 50000
E = 800000
N_PAD = 50176
ROWS_PER_TILE = N_PAD // 16
ZROWS = ROWS_PER_TILE // 16
K = 128
S = 2
E_PAD = 819200
TILE_ROWS = E_PAD // (16 * K)
OUTER = TILE_ROWS // S
EDGES_PER_TILE = E // 16
DEG_CHUNK = 2000

_MESH = plsc.VectorSubcoreMesh(core_axis_name="c", subcore_axis_name="s")
_SC_PARAMS = pltpu.CompilerParams(use_tc_tiling_on_sc=False,
                                  needs_layout_passes=False)


@functools.partial(
    pl.kernel,
    out_type=jax.ShapeDtypeStruct((2 * 16 * N_PAD,), jnp.float32),
    mesh=_MESH,
    scratch_types=[
        pltpu.VMEM((N_PAD,), jnp.float32),
        pltpu.VMEM((DEG_CHUNK,), jnp.int32),
    ],
    compiler_params=_SC_PARAMS,
)
def _degree_kernel(src_ref, dst_ref, out_ref, cnt, idxbuf):
    c = lax.axis_index("c")
    s = lax.axis_index("s")
    z = jnp.zeros((16,), jnp.float32)
    ones = jnp.ones((16,), jnp.float32)

    @pl.loop(0, N_PAD // 16)
    def _zero(i):
        cnt[pl.ds(i * 16, 16)] = z

    def count(idx_ref):
        @pl.loop(0, EDGES_PER_TILE // DEG_CHUNK)
        def _outer(o):
            base = pl.multiple_of(s * EDGES_PER_TILE + o * DEG_CHUNK, 8)
            pltpu.sync_copy(idx_ref.at[pl.ds(base, DEG_CHUNK)], idxbuf)

            @pl.loop(0, DEG_CHUNK // 16)
            def _inner(i):
                idxv = idxbuf[pl.ds(i * 16, 16)]
                plsc.addupdate_scatter(cnt, [idxv], ones)

    @pl.when(c == 0)
    def _():
        count(src_ref)

    @pl.when(c == 1)
    def _():
        count(dst_ref)

    obase = pl.multiple_of((c * 16 + s) * N_PAD, 128)
    pltpu.sync_copy(cnt, out_ref.at[pl.ds(obase, N_PAD)])


def _make_agg(n_chunks):
    @functools.partial(
        pl.kernel,
        out_type=jax.ShapeDtypeStruct((n_chunks, N_PAD, 32), jnp.float32),
        mesh=_MESH,
        scratch_types=[
            pltpu.VMEM_SHARED((N_PAD, 32), jnp.float32),
            pltpu.VMEM((3, S, K), jnp.int32),
            pltpu.VMEM((3, S, K), jnp.int32),
            pltpu.VMEM((2, S, K, 32), jnp.float32),
            pltpu.VMEM((ZROWS, 32), jnp.float32),
            pltpu.SemaphoreType.DMA,
            pltpu.SemaphoreType.DMA,
            pltpu.SemaphoreType.DMA,
        ],
        compiler_params=_SC_PARAMS,
    )
    def _agg(src_ref, dst_ref, h_all, out_ref, acc, srcbuf, dstbuf, msg,
             zbuf, gsem, ssem, isem):
        c = lax.axis_index("c")
        s = lax.axis_index("s")
        z = jnp.zeros((16,), jnp.float32)
        zi = jnp.zeros((16,), jnp.int32)

        @pl.loop(0, ZROWS)
        def _zfill(i):
            zbuf[i, pl.ds(0, 16)] = z
            zbuf[i, pl.ds(16, 16)] = z

        for j in range(S):
            @pl.loop(0, K // 16)
            def _dzero(i, j=j):
                dstbuf[0, j, pl.ds(i * 16, 16)] = zi

        def do_chunk(h_ref, chunk):
            rbase = pl.multiple_of(s * ROWS_PER_TILE, 8)

            @pl.loop(0, ROWS_PER_TILE // ZROWS)
            def _za(j):
                pltpu.sync_copy(zbuf, acc.at[pl.ds(rbase + j * ZROWS, ZROWS)])
            plsc.subcore_barrier()

            for j in range(S):
                pltpu.async_copy(zbuf.at[pl.ds(0, K)],
                                 acc.at[dstbuf.at[0, j]], ssem, add=True)
            row0 = s * TILE_ROWS
            pltpu.sync_copy(src_ref.at[pl.ds(row0, S)], srcbuf.at[0])
            pltpu.sync_copy(dst_ref.at[pl.ds(row0, S)], dstbuf.at[0])

            @pl.loop(0, OUTER)
            def _edges(o):
                ib = lax.rem(o, 3)
                nib = lax.rem(o + 1, 3)
                mb = lax.rem(o, 2)

                @pl.when(o > 0)
                def _():
                    pltpu.make_async_copy(
                        src_ref.at[pl.ds(row0, S)], srcbuf.at[0], isem).wait()
                    pltpu.make_async_copy(
                        dst_ref.at[pl.ds(row0, S)], dstbuf.at[0], isem).wait()

                gs = [pltpu.async_copy(h_ref.at[srcbuf.at[ib, j]],
                                       msg.at[mb, j], gsem)
                      for j in range(S)]

                @pl.when(o < OUTER - 1)
                def _():
                    rown = s * TILE_ROWS + (o + 1) * S
                    pltpu.async_copy(src_ref.at[pl.ds(rown, S)],
                                     srcbuf.at[nib], isem)
                    pltpu.async_copy(dst_ref.at[pl.ds(rown, S)],
                                     dstbuf.at[nib], isem)

                for g in gs:
                    g.wait()
                for j in range(S):
                    pltpu.make_async_copy(
                        msg.at[0, j], acc.at[dstbuf.at[0, j]], ssem).wait()
                for j in range(S):
                    pltpu.async_copy(msg.at[mb, j], acc.at[dstbuf.at[ib, j]],
                                     ssem, add=True)

            for j in range(S):
                pltpu.make_async_copy(
                    msg.at[0, j], acc.at[dstbuf.at[0, j]], ssem).wait()

            plsc.subcore_barrier()
            pltpu.sync_copy(
                acc.at[pl.ds(rbase, ROWS_PER_TILE)],
                out_ref.at[chunk, pl.ds(rbase, ROWS_PER_TILE)])

        for phase in range(n_chunks // 2):
            for core in range(2):
                chunk = 2 * phase + core

                @pl.when(c == core)
                def _(chunk=chunk):
                    do_chunk(h_all.at[chunk], chunk)

    return _agg


_agg4 = _make_agg(4)
_agg2 = _make_agg(2)


BLK = 1024
GRID = N_PAD // BLK


def _isqrts(pblk):
    deg = jnp.maximum(jnp.sum(pblk, axis=1), 1.0)
    return lax.rsqrt(deg)


def _split_store(out_ref, m):
    for q in range(out_ref.shape[0]):
        out_ref[q] = m[:, q * 32:(q + 1) * 32]


def _cat_chunks(a_ref):
    return jnp.concatenate([a_ref[q] for q in range(a_ref.shape[0])], axis=1)


def _stage1_body(feat_ref, p_ref, w1_ref, h1_ref):
    sc = _isqrts(p_ref[...])
    x = feat_ref[...] * sc[0][:, None]
    _split_store(h1_ref,
                 jnp.dot(x, w1_ref[...], preferred_element_type=jnp.float32))


_stage1 = pl.pallas_call(
    _stage1_body,
    grid=(GRID,),
    in_specs=[
        pl.BlockSpec((BLK, 128), lambda i: (i, 0)),
        pl.BlockSpec((2, 16, BLK), lambda i: (0, 0, i)),
        pl.BlockSpec((128, 128), lambda i: (0, 0)),
    ],
    out_specs=pl.BlockSpec((4, BLK, 32), lambda i: (0, i, 0)),
    out_shape=jax.ShapeDtypeStruct((4, N_PAD, 32), jnp.float32),
)


def _stage2_body(agg_ref, p_ref, b1_ref, w2_ref, h2_ref):
    sc = _isqrts(p_ref[...])
    x1 = jnp.maximum(_cat_chunks(agg_ref) * sc[1][:, None] + b1_ref[...], 0.0)
    _split_store(h2_ref,
                 jnp.dot(x1 * sc[0][:, None], w2_ref[...],
                         preferred_element_type=jnp.float32))


_stage2 = pl.pallas_call(
    _stage2_body,
    grid=(GRID,),
    in_specs=[
        pl.BlockSpec((4, BLK, 32), lambda i: (0, i, 0)),
        pl.BlockSpec((2, 16, BLK), lambda i: (0, 0, i)),
        pl.BlockSpec((1, 128), lambda i: (0, 0)),
        pl.BlockSpec((128, 64), lambda i: (0, 0)),
    ],
    out_specs=pl.BlockSpec((2, BLK, 32), lambda i: (0, i, 0)),
    out_shape=jax.ShapeDtypeStruct((2, N_PAD, 32), jnp.float32),
)


def _stage3_body(agg_ref, p_ref, b2_ref, wo1_ref, bo1_ref, wo2_ref, bo2_ref,
                 hid_ref, log_ref):
    sc = _isqrts(p_ref[...])
    hid = _cat_chunks(agg_ref) * sc[1][:, None] + b2_ref[...]
    hid_ref[...] = hid
    t = jnp.maximum(
        jnp.dot(hid, wo1_ref[...], preferred_element_type=jnp.float32)
        + bo1_ref[...], 0.0)
    log_ref[...] = jnp.dot(t, wo2_ref[...],
                           preferred_element_type=jnp.float32) + bo2_ref[...]


_stage3 = pl.pallas_call(
    _stage3_body,
    grid=(GRID,),
    in_specs=[
        pl.BlockSpec((2, BLK, 32), lambda i: (0, i, 0)),
        pl.BlockSpec((2, 16, BLK), lambda i: (0, 0, i)),
        pl.BlockSpec((1, 64), lambda i: (0, 0)),
        pl.BlockSpec((64, 128), lambda i: (0, 0)),
        pl.BlockSpec((1, 128), lambda i: (0, 0)),
        pl.BlockSpec((128, 128), lambda i: (0, 0)),
        pl.BlockSpec((1, 128), lambda i: (0, 0)),
    ],
    out_specs=[
        pl.BlockSpec((BLK, 64), lambda i: (i, 0)),
        pl.BlockSpec((BLK, 128), lambda i: (i, 0)),
    ],
    out_shape=[
        jax.ShapeDtypeStruct((N_PAD, 64), jnp.float32),
        jax.ShapeDtypeStruct((N_PAD, 128), jnp.float32),
    ],
)


def kernel(features, edge_index, W1, b1, W2, b2, Wo1, bo1, Wo2, bo2):
    n = features.shape[0]
    feat = features.reshape(n, -1)
    featp = jnp.pad(feat, ((0, N_PAD - n), (0, 0)))
    src = edge_index[0]
    dst = edge_index[1]
    npad_e = E_PAD - E
    src2d = jnp.concatenate(
        [src, jnp.zeros((npad_e,), jnp.int32)]).reshape(-1, K)
    dst2d = jnp.concatenate(
        [dst, jnp.full((npad_e,), N_PAD - 1, jnp.int32)]).reshape(-1, K)

    partials = _degree_kernel(src, dst).reshape(2, 16, N_PAD)

    h1t = _stage1(featp, partials, W1)
    agg1t = _agg4(src2d, dst2d, h1t)
    h2t = _stage2(agg1t, partials, b1.reshape(1, -1), W2)
    agg2t = _agg2(src2d, dst2d, h2t)

    wo2p = jnp.pad(Wo2, ((0, 0), (0, 128 - Wo2.shape[1])))
    bo2p = jnp.pad(bo2, (0, 128 - bo2.shape[0])).reshape(1, -1)
    hid_p, log_p = _stage3(agg2t, partials, b2.reshape(1, -1), Wo1,
                           bo1.reshape(1, -1), wo2p, bo2p)
    logits = log_p[:n, :2]
    hidden = hid_p[:n]
    return (logits, logits, hidden)


# speedup vs baseline: 1.0375x; 1.0007x over previous
import functools

import jax
import jax.numpy as jnp
from jax import lax
from jax.experimental import pallas as pl
from jax.experimental.pallas import tpu as pltpu
from jax.experimental.pallas import tpu_sc as plsc

N = 50000
E = 800000
N_PAD = 50176
ROWS_PER_TILE = N_PAD // 16
ZROWS = ROWS_PER_TILE // 16
K = 128
S = 2
E_PAD = 819200
TILE_ROWS = E_PAD // (16 * K)
OUTER = TILE_ROWS // S
EDGES_PER_TILE = E // 16
DEG_CHUNK = 2000

_MESH = plsc.VectorSubcoreMesh(core_axis_name="c", subcore_axis_name="s")
_SC_PARAMS = pltpu.CompilerParams(use_tc_tiling_on_sc=False,
                                  needs_layout_passes=False)


PAD_PER_TILE = (E_PAD - E) // 16


@functools.partial(
    pl.kernel,
    out_type=[
        jax.ShapeDtypeStruct((2 * 16 * N_PAD,), jnp.float32),
        jax.ShapeDtypeStruct((E_PAD,), jnp.int32),
        jax.ShapeDtypeStruct((E_PAD,), jnp.int32),
    ],
    mesh=_MESH,
    scratch_types=[
        pltpu.VMEM((N_PAD,), jnp.float32),
        pltpu.VMEM((DEG_CHUNK,), jnp.int32),
    ],
    compiler_params=_SC_PARAMS,
)
def _degree_kernel(src_ref, dst_ref, out_ref, srcp_ref, dstp_ref, cnt, idxbuf):
    c = lax.axis_index("c")
    s = lax.axis_index("s")
    z = jnp.zeros((16,), jnp.float32)
    ones = jnp.ones((16,), jnp.float32)

    @pl.loop(0, N_PAD // 16)
    def _zero(i):
        cnt[pl.ds(i * 16, 16)] = z

    def count(idx_ref, pad_ref, padval):
        @pl.loop(0, EDGES_PER_TILE // DEG_CHUNK)
        def _outer(o):
            base = pl.multiple_of(s * EDGES_PER_TILE + o * DEG_CHUNK, 8)
            pltpu.sync_copy(idx_ref.at[pl.ds(base, DEG_CHUNK)], idxbuf)
            pltpu.sync_copy(idxbuf, pad_ref.at[pl.ds(base, DEG_CHUNK)])

            @pl.loop(0, DEG_CHUNK // 16)
            def _inner(i):
                idxv = idxbuf[pl.ds(i * 16, 16)]
                plsc.addupdate_scatter(cnt, [idxv], ones)

        @pl.loop(0, PAD_PER_TILE // 16)
        def _fill(i):
            idxbuf[pl.ds(i * 16, 16)] = padval

        pbase = pl.multiple_of(E + s * PAD_PER_TILE, 8)
        pltpu.sync_copy(idxbuf.at[pl.ds(0, PAD_PER_TILE)],
                        pad_ref.at[pl.ds(pbase, PAD_PER_TILE)])

    @pl.when(c == 0)
    def _():
        count(src_ref, srcp_ref, jnp.zeros((16,), jnp.int32))

    @pl.when(c == 1)
    def _():
        count(dst_ref, dstp_ref, jnp.full((16,), N_PAD - 1, jnp.int32))

    obase = pl.multiple_of((c * 16 + s) * N_PAD, 128)
    pltpu.sync_copy(cnt, out_ref.at[pl.ds(obase, N_PAD)])


def _make_agg(n_chunks):
    @functools.partial(
        pl.kernel,
        out_type=jax.ShapeDtypeStruct((n_chunks, N_PAD, 32), jnp.float32),
        mesh=_MESH,
        scratch_types=[
            pltpu.VMEM_SHARED((N_PAD, 32), jnp.float32),
            pltpu.VMEM((3, S, K), jnp.int32),
            pltpu.VMEM((3, S, K), jnp.int32),
            pltpu.VMEM((2, S, K, 32), jnp.float32),
            pltpu.VMEM((ZROWS, 32), jnp.float32),
            pltpu.SemaphoreType.DMA,
            pltpu.SemaphoreType.DMA,
            pltpu.SemaphoreType.DMA,
        ],
        compiler_params=_SC_PARAMS,
    )
    def _agg(src_ref, dst_ref, h_all, out_ref, acc, srcbuf, dstbuf, msg,
             zbuf, gsem, ssem, isem):
        c = lax.axis_index("c")
        s = lax.axis_index("s")
        z = jnp.zeros((16,), jnp.float32)
        zi = jnp.zeros((16,), jnp.int32)

        @pl.loop(0, ZROWS)
        def _zfill(i):
            zbuf[i, pl.ds(0, 16)] = z
            zbuf[i, pl.ds(16, 16)] = z

        for j in range(S):
            @pl.loop(0, K // 16)
            def _dzero(i, j=j):
                dstbuf[0, j, pl.ds(i * 16, 16)] = zi

        def do_chunk(h_ref, chunk):
            rbase = pl.multiple_of(s * ROWS_PER_TILE, 8)

            @pl.loop(0, ROWS_PER_TILE // ZROWS)
            def _za(j):
                pltpu.sync_copy(zbuf, acc.at[pl.ds(rbase + j * ZROWS, ZROWS)])
            plsc.subcore_barrier()

            for j in range(S):
                pltpu.async_copy(zbuf.at[pl.ds(0, K)],
                                 acc.at[dstbuf.at[0, j]], ssem, add=True)
            row0 = s * TILE_ROWS
            pltpu.sync_copy(src_ref.at[pl.ds(row0, S)], srcbuf.at[0])
            pltpu.sync_copy(dst_ref.at[pl.ds(row0, S)], dstbuf.at[0])

            @pl.loop(0, OUTER)
            def _edges(o):
                ib = lax.rem(o, 3)
                nib = lax.rem(o + 1, 3)
                mb = lax.rem(o, 2)

                @pl.when(o > 0)
                def _():
                    pltpu.make_async_copy(
                        src_ref.at[pl.ds(row0, S)], srcbuf.at[0], isem).wait()
                    pltpu.make_async_copy(
                        dst_ref.at[pl.ds(row0, S)], dstbuf.at[0], isem).wait()

                gs = [pltpu.async_copy(h_ref.at[srcbuf.at[ib, j]],
                                       msg.at[mb, j], gsem)
                      for j in range(S)]

                @pl.when(o < OUTER - 1)
                def _():
                    rown = s * TILE_ROWS + (o + 1) * S
                    pltpu.async_copy(src_ref.at[pl.ds(rown, S)],
                                     srcbuf.at[nib], isem)
                    pltpu.async_copy(dst_ref.at[pl.ds(rown, S)],
                                     dstbuf.at[nib], isem)

                for g in gs:
                    g.wait()
                for j in range(S):
                    pltpu.make_async_copy(
                        msg.at[0, j], acc.at[dstbuf.at[0, j]], ssem).wait()
                for j in range(S):
                    pltpu.async_copy(msg.at[mb, j], acc.at[dstbuf.at[ib, j]],
                                     ssem, add=True)

            for j in range(S):
                pltpu.make_async_copy(
                    msg.at[0, j], acc.at[dstbuf.at[0, j]], ssem).wait()

            plsc.subcore_barrier()
            pltpu.sync_copy(
                acc.at[pl.ds(rbase, ROWS_PER_TILE)],
                out_ref.at[chunk, pl.ds(rbase, ROWS_PER_TILE)])

        for phase in range(n_chunks // 2):
            for core in range(2):
                chunk = 2 * phase + core

                @pl.when(c == core)
                def _(chunk=chunk):
                    do_chunk(h_all.at[chunk], chunk)

    return _agg


_agg4 = _make_agg(4)
_agg2 = _make_agg(2)


BLK = 1024
GRID = N_PAD // BLK


def _isqrts(pblk):
    deg = jnp.maximum(jnp.sum(pblk, axis=1), 1.0)
    return lax.rsqrt(deg)


def _split_store(out_ref, m):
    for q in range(out_ref.shape[0]):
        out_ref[q] = m[:, q * 32:(q + 1) * 32]


def _cat_chunks(a_ref):
    return jnp.concatenate([a_ref[q] for q in range(a_ref.shape[0])], axis=1)


def _stage1_body(feat_ref, p_ref, w1_ref, h1_ref):
    sc = _isqrts(p_ref[...])
    x = feat_ref[...] * sc[0][:, None]
    _split_store(h1_ref,
                 jnp.dot(x, w1_ref[...], preferred_element_type=jnp.float32))


_stage1 = pl.pallas_call(
    _stage1_body,
    grid=(GRID,),
    in_specs=[
        pl.BlockSpec((BLK, 128), lambda i: (i, 0)),
        pl.BlockSpec((2, 16, BLK), lambda i: (0, 0, i)),
        pl.BlockSpec((128, 128), lambda i: (0, 0)),
    ],
    out_specs=pl.BlockSpec((4, BLK, 32), lambda i: (0, i, 0)),
    out_shape=jax.ShapeDtypeStruct((4, N_PAD, 32), jnp.float32),
)


def _stage2_body(agg_ref, p_ref, b1_ref, w2_ref, h2_ref):
    sc = _isqrts(p_ref[...])
    x1 = jnp.maximum(_cat_chunks(agg_ref) * sc[1][:, None] + b1_ref[...], 0.0)
    _split_store(h2_ref,
                 jnp.dot(x1 * sc[0][:, None], w2_ref[...],
                         preferred_element_type=jnp.float32))


_stage2 = pl.pallas_call(
    _stage2_body,
    grid=(GRID,),
    in_specs=[
        pl.BlockSpec((4, BLK, 32), lambda i: (0, i, 0)),
        pl.BlockSpec((2, 16, BLK), lambda i: (0, 0, i)),
        pl.BlockSpec((1, 128), lambda i: (0, 0)),
        pl.BlockSpec((128, 64), lambda i: (0, 0)),
    ],
    out_specs=pl.BlockSpec((2, BLK, 32), lambda i: (0, i, 0)),
    out_shape=jax.ShapeDtypeStruct((2, N_PAD, 32), jnp.float32),
)


def _stage3_body(agg_ref, p_ref, b2_ref, wo1_ref, bo1_ref, wo2_ref, bo2_ref,
                 hid_ref, log_ref):
    sc = _isqrts(p_ref[...])
    hid = _cat_chunks(agg_ref) * sc[1][:, None] + b2_ref[...]
    hid_ref[...] = hid
    t = jnp.maximum(
        jnp.dot(hid, wo1_ref[...], preferred_element_type=jnp.float32)
        + bo1_ref[...], 0.0)
    log_ref[...] = jnp.dot(t, wo2_ref[...],
                           preferred_element_type=jnp.float32) + bo2_ref[...]


_stage3 = pl.pallas_call(
    _stage3_body,
    grid=(GRID,),
    in_specs=[
        pl.BlockSpec((2, BLK, 32), lambda i: (0, i, 0)),
        pl.BlockSpec((2, 16, BLK), lambda i: (0, 0, i)),
        pl.BlockSpec((1, 64), lambda i: (0, 0)),
        pl.BlockSpec((64, 128), lambda i: (0, 0)),
        pl.BlockSpec((1, 128), lambda i: (0, 0)),
        pl.BlockSpec((128, 128), lambda i: (0, 0)),
        pl.BlockSpec((1, 128), lambda i: (0, 0)),
    ],
    out_specs=[
        pl.BlockSpec((BLK, 64), lambda i: (i, 0)),
        pl.BlockSpec((BLK, 128), lambda i: (i, 0)),
    ],
    out_shape=[
        jax.ShapeDtypeStruct((N, 64), jnp.float32),
        jax.ShapeDtypeStruct((N, 128), jnp.float32),
    ],
)


def kernel(features, edge_index, W1, b1, W2, b2, Wo1, bo1, Wo2, bo2):
    n = features.shape[0]
    feat = features.reshape(n, -1)
    src = edge_index[0]
    dst = edge_index[1]

    partials_flat, srcp, dstp = _degree_kernel(src, dst)
    partials = partials_flat.reshape(2, 16, N_PAD)
    src2d = srcp.reshape(-1, K)
    dst2d = dstp.reshape(-1, K)

    h1t = _stage1(feat, partials, W1)
    agg1t = _agg4(src2d, dst2d, h1t)
    h2t = _stage2(agg1t, partials, b1.reshape(1, -1), W2)
    agg2t = _agg2(src2d, dst2d, h2t)

    wo2p = jnp.pad(Wo2, ((0, 0), (0, 128 - Wo2.shape[1])))
    bo2p = jnp.pad(bo2, (0, 128 - bo2.shape[0])).reshape(1, -1)
    hidden, log_p = _stage3(agg2t, partials, b2.reshape(1, -1), Wo1,
                            bo1.reshape(1, -1), wo2p, bo2p)
    logits = log_p[:, :2]
    return (logits, logits, hidden)


# speedup vs baseline: 1.1187x; 1.0783x over previous
import functools

import jax
import jax.numpy as jnp
from jax import lax
from jax.experimental import pallas as pl
from jax.experimental.pallas import tpu as pltpu
from jax.experimental.pallas import tpu_sc as plsc

N = 50000
E = 800000
N_PAD = 50176
ROWS_PER_TILE = N_PAD // 16
ZROWS = ROWS_PER_TILE // 64
K = 128
S = 2
E_PAD = 819200
TILE_ROWS = E_PAD // (16 * K)
OUTER = TILE_ROWS // S
EDGES_PER_TILE = E // 16
DEG_CHUNK = 2000

_MESH = plsc.VectorSubcoreMesh(core_axis_name="c", subcore_axis_name="s")
_SC_PARAMS = pltpu.CompilerParams(use_tc_tiling_on_sc=False,
                                  needs_layout_passes=False)


PAD_PER_TILE = (E_PAD - E) // 16


@functools.partial(
    pl.kernel,
    out_type=[
        jax.ShapeDtypeStruct((2 * 16 * N_PAD,), jnp.float32),
        jax.ShapeDtypeStruct((E_PAD,), jnp.int32),
        jax.ShapeDtypeStruct((E_PAD,), jnp.int32),
    ],
    mesh=_MESH,
    scratch_types=[
        pltpu.VMEM((N_PAD,), jnp.float32),
        pltpu.VMEM((DEG_CHUNK,), jnp.int32),
    ],
    compiler_params=_SC_PARAMS,
)
def _degree_kernel(src_ref, dst_ref, out_ref, srcp_ref, dstp_ref, cnt, idxbuf):
    c = lax.axis_index("c")
    s = lax.axis_index("s")
    z = jnp.zeros((16,), jnp.float32)
    ones = jnp.ones((16,), jnp.float32)

    @pl.loop(0, N_PAD // 16)
    def _zero(i):
        cnt[pl.ds(i * 16, 16)] = z

    def count(idx_ref, pad_ref, padval):
        @pl.loop(0, EDGES_PER_TILE // DEG_CHUNK)
        def _outer(o):
            base = pl.multiple_of(s * EDGES_PER_TILE + o * DEG_CHUNK, 8)
            pltpu.sync_copy(idx_ref.at[pl.ds(base, DEG_CHUNK)], idxbuf)
            pltpu.sync_copy(idxbuf, pad_ref.at[pl.ds(base, DEG_CHUNK)])

            @pl.loop(0, DEG_CHUNK // 16)
            def _inner(i):
                idxv = idxbuf[pl.ds(i * 16, 16)]
                plsc.addupdate_scatter(cnt, [idxv], ones)

        @pl.loop(0, PAD_PER_TILE // 16)
        def _fill(i):
            idxbuf[pl.ds(i * 16, 16)] = padval

        pbase = pl.multiple_of(E + s * PAD_PER_TILE, 8)
        pltpu.sync_copy(idxbuf.at[pl.ds(0, PAD_PER_TILE)],
                        pad_ref.at[pl.ds(pbase, PAD_PER_TILE)])

    @pl.when(c == 0)
    def _():
        count(src_ref, srcp_ref, jnp.zeros((16,), jnp.int32))

    @pl.when(c == 1)
    def _():
        count(dst_ref, dstp_ref, jnp.full((16,), N_PAD - 1, jnp.int32))

    obase = pl.multiple_of((c * 16 + s) * N_PAD, 128)
    pltpu.sync_copy(cnt, out_ref.at[pl.ds(obase, N_PAD)])


def _make_agg(n_chunks):
    @functools.partial(
        pl.kernel,
        out_type=jax.ShapeDtypeStruct((n_chunks, N_PAD, 32), jnp.float32),
        mesh=_MESH,
        scratch_types=[
            pltpu.VMEM_SHARED((N_PAD, 32), jnp.float32),
            pltpu.VMEM((4, S, K), jnp.int32),
            pltpu.VMEM((4, S, K), jnp.int32),
            pltpu.VMEM((3, S, K, 32), jnp.float32),
            pltpu.VMEM((ZROWS, 32), jnp.float32),
            pltpu.SemaphoreType.DMA,
            pltpu.SemaphoreType.DMA,
            pltpu.SemaphoreType.DMA,
            pltpu.SemaphoreType.DMA,
        ],
        compiler_params=_SC_PARAMS,
    )
    def _agg(src_ref, dst_ref, h_all, out_ref, acc, srcbuf, dstbuf, msg,
             zbuf, gsem_e, gsem_o, ssem, isem):
        c = lax.axis_index("c")
        s = lax.axis_index("s")
        z = jnp.zeros((16,), jnp.float32)
        zi = jnp.zeros((16,), jnp.int32)

        @pl.loop(0, ZROWS)
        def _zfill(i):
            zbuf[i, pl.ds(0, 16)] = z
            zbuf[i, pl.ds(16, 16)] = z

        for j in range(S):
            @pl.loop(0, K // 16)
            def _dzero(i, j=j):
                dstbuf[0, j, pl.ds(i * 16, 16)] = zi

        def do_chunk(h_ref, chunk):
            rbase = pl.multiple_of(s * ROWS_PER_TILE, 8)

            @pl.loop(0, ROWS_PER_TILE // ZROWS)
            def _za(j):
                pltpu.sync_copy(zbuf, acc.at[pl.ds(rbase + j * ZROWS, ZROWS)])
            plsc.subcore_barrier()

            for j in range(S):
                pltpu.async_copy(zbuf.at[pl.ds(0, K)],
                                 acc.at[dstbuf.at[0, j]], ssem, add=True)
            row0 = s * TILE_ROWS
            pltpu.sync_copy(src_ref.at[pl.ds(row0, S)], srcbuf.at[0])
            pltpu.sync_copy(dst_ref.at[pl.ds(row0, S)], dstbuf.at[0])
            pltpu.sync_copy(src_ref.at[pl.ds(row0 + S, S)], srcbuf.at[1])
            pltpu.sync_copy(dst_ref.at[pl.ds(row0 + S, S)], dstbuf.at[1])
            for j in range(S):
                pltpu.async_copy(h_ref.at[srcbuf.at[0, j]], msg.at[0, j],
                                 gsem_e)

            @pl.loop(0, OUTER)
            def _edges(o):
                ib = lax.rem(o, 4)
                ib1 = lax.rem(o + 1, 4)
                ib2 = lax.rem(o + 2, 4)
                mb = lax.rem(o, 3)
                mb1 = lax.rem(o + 1, 3)

                @pl.when(jnp.logical_and(o > 0, o < OUTER - 1))
                def _():
                    pltpu.make_async_copy(
                        src_ref.at[pl.ds(row0, S)], srcbuf.at[0], isem).wait()
                    pltpu.make_async_copy(
                        dst_ref.at[pl.ds(row0, S)], dstbuf.at[0], isem).wait()

                def fire_next(sem):
                    @pl.when(o < OUTER - 1)
                    def _():
                        for j in range(S):
                            pltpu.async_copy(h_ref.at[srcbuf.at[ib1, j]],
                                             msg.at[mb1, j], sem)

                def drain_cur(sem):
                    for j in range(S):
                        pltpu.make_async_copy(h_ref.at[srcbuf.at[0, j]],
                                              msg.at[0, j], sem).wait()

                @pl.when(lax.rem(o, 2) == 0)
                def _():
                    fire_next(gsem_o)
                    drain_cur(gsem_e)

                @pl.when(lax.rem(o, 2) == 1)
                def _():
                    fire_next(gsem_e)
                    drain_cur(gsem_o)

                @pl.when(o < OUTER - 2)
                def _():
                    rown = s * TILE_ROWS + (o + 2) * S
                    pltpu.async_copy(src_ref.at[pl.ds(rown, S)],
                                     srcbuf.at[ib2], isem)
                    pltpu.async_copy(dst_ref.at[pl.ds(rown, S)],
                                     dstbuf.at[ib2], isem)

                for j in range(S):
                    pltpu.make_async_copy(
                        msg.at[0, j], acc.at[dstbuf.at[0, j]], ssem).wait()
                for j in range(S):
                    pltpu.async_copy(msg.at[mb, j], acc.at[dstbuf.at[ib, j]],
                                     ssem, add=True)

            for j in range(S):
                pltpu.make_async_copy(
                    msg.at[0, j], acc.at[dstbuf.at[0, j]], ssem).wait()

            plsc.subcore_barrier()
            pltpu.sync_copy(
                acc.at[pl.ds(rbase, ROWS_PER_TILE)],
                out_ref.at[chunk, pl.ds(rbase, ROWS_PER_TILE)])

        for phase in range(n_chunks // 2):
            for core in range(2):
                chunk = 2 * phase + core

                @pl.when(c == core)
                def _(chunk=chunk):
                    do_chunk(h_all.at[chunk], chunk)

    return _agg


_agg4 = _make_agg(4)
_agg2 = _make_agg(2)


BLK = 1024
GRID = N_PAD // BLK


def _isqrts(pblk):
    deg = jnp.maximum(jnp.sum(pblk, axis=1), 1.0)
    return lax.rsqrt(deg)


def _split_store(out_ref, m):
    for q in range(out_ref.shape[0]):
        out_ref[q] = m[:, q * 32:(q + 1) * 32]


def _cat_chunks(a_ref):
    return jnp.concatenate([a_ref[q] for q in range(a_ref.shape[0])], axis=1)


def _stage1_body(feat_ref, p_ref, w1_ref, h1_ref):
    sc = _isqrts(p_ref[...])
    x = feat_ref[...] * sc[0][:, None]
    _split_store(h1_ref,
                 jnp.dot(x, w1_ref[...], preferred_element_type=jnp.float32))


_stage1 = pl.pallas_call(
    _stage1_body,
    grid=(GRID,),
    in_specs=[
        pl.BlockSpec((BLK, 128), lambda i: (i, 0)),
        pl.BlockSpec((2, 16, BLK), lambda i: (0, 0, i)),
        pl.BlockSpec((128, 128), lambda i: (0, 0)),
    ],
    out_specs=pl.BlockSpec((4, BLK, 32), lambda i: (0, i, 0)),
    out_shape=jax.ShapeDtypeStruct((4, N_PAD, 32), jnp.float32),
)


def _stage2_body(agg_ref, p_ref, b1_ref, w2_ref, h2_ref):
    sc = _isqrts(p_ref[...])
    x1 = jnp.maximum(_cat_chunks(agg_ref) * sc[1][:, None] + b1_ref[...], 0.0)
    _split_store(h2_ref,
                 jnp.dot(x1 * sc[0][:, None], w2_ref[...],
                         preferred_element_type=jnp.float32))


_stage2 = pl.pallas_call(
    _stage2_body,
    grid=(GRID,),
    in_specs=[
        pl.BlockSpec((4, BLK, 32), lambda i: (0, i, 0)),
        pl.BlockSpec((2, 16, BLK), lambda i: (0, 0, i)),
        pl.BlockSpec((1, 128), lambda i: (0, 0)),
        pl.BlockSpec((128, 64), lambda i: (0, 0)),
    ],
    out_specs=pl.BlockSpec((2, BLK, 32), lambda i: (0, i, 0)),
    out_shape=jax.ShapeDtypeStruct((2, N_PAD, 32), jnp.float32),
)


def _stage3_body(agg_ref, p_ref, b2_ref, wo1_ref, bo1_ref, wo2_ref, bo2_ref,
                 hid_ref, log_ref):
    sc = _isqrts(p_ref[...])
    hid = _cat_chunks(agg_ref) * sc[1][:, None] + b2_ref[...]
    hid_ref[...] = hid
    t = jnp.maximum(
        jnp.dot(hid, wo1_ref[...], preferred_element_type=jnp.float32)
        + bo1_ref[...], 0.0)
    log_ref[...] = jnp.dot(t, wo2_ref[...],
                           preferred_element_type=jnp.float32) + bo2_ref[...]


_stage3 = pl.pallas_call(
    _stage3_body,
    grid=(GRID,),
    in_specs=[
        pl.BlockSpec((2, BLK, 32), lambda i: (0, i, 0)),
        pl.BlockSpec((2, 16, BLK), lambda i: (0, 0, i)),
        pl.BlockSpec((1, 64), lambda i: (0, 0)),
        pl.BlockSpec((64, 128), lambda i: (0, 0)),
        pl.BlockSpec((1, 128), lambda i: (0, 0)),
        pl.BlockSpec((128, 128), lambda i: (0, 0)),
        pl.BlockSpec((1, 128), lambda i: (0, 0)),
    ],
    out_specs=[
        pl.BlockSpec((BLK, 64), lambda i: (i, 0)),
        pl.BlockSpec((BLK, 128), lambda i: (i, 0)),
    ],
    out_shape=[
        jax.ShapeDtypeStruct((N, 64), jnp.float32),
        jax.ShapeDtypeStruct((N, 128), jnp.float32),
    ],
)


def kernel(features, edge_index, W1, b1, W2, b2, Wo1, bo1, Wo2, bo2):
    n = features.shape[0]
    feat = features.reshape(n, -1)
    src = edge_index[0]
    dst = edge_index[1]

    partials_flat, srcp, dstp = _degree_kernel(src, dst)
    partials = partials_flat.reshape(2, 16, N_PAD)
    src2d = srcp.reshape(-1, K)
    dst2d = dstp.reshape(-1, K)

    h1t = _stage1(feat, partials, W1)
    agg1t = _agg4(src2d, dst2d, h1t)
    h2t = _stage2(agg1t, partials, b1.reshape(1, -1), W2)
    agg2t = _agg2(src2d, dst2d, h2t)

    wo2p = jnp.pad(Wo2, ((0, 0), (0, 128 - Wo2.shape[1])))
    bo2p = jnp.pad(bo2, (0, 128 - bo2.shape[0])).reshape(1, -1)
    hidden, log_p = _stage3(agg2t, partials, b2.reshape(1, -1), Wo1,
                            bo1.reshape(1, -1), wo2p, bo2p)
    logits = log_p[:, :2]
    return (logits, logits, hidden)


# speedup vs baseline: 1.1264x; 1.0069x over previous
import functools

import jax
import jax.numpy as jnp
from jax import lax
from jax.experimental import pallas as pl
from jax.experimental.pallas import tpu as pltpu
from jax.experimental.pallas import tpu_sc as plsc

N = 50000
E = 800000
N_PAD = 50176
ROWS_PER_TILE = N_PAD // 16
ZROWS = ROWS_PER_TILE // 64
K = 128
S = 2
E_PAD = 819200
TILE_ROWS = E_PAD // (16 * K)
OUTER = TILE_ROWS // S
EDGES_PER_TILE = E // 16
DEG_CHUNK = 2000

_MESH = plsc.VectorSubcoreMesh(core_axis_name="c", subcore_axis_name="s")
_SC_PARAMS = pltpu.CompilerParams(use_tc_tiling_on_sc=False,
                                  needs_layout_passes=False)


PAD_PER_TILE = (E_PAD - E) // 16


@functools.partial(
    pl.kernel,
    out_type=[
        jax.ShapeDtypeStruct((2 * 16 * N_PAD,), jnp.float32),
        jax.ShapeDtypeStruct((E_PAD,), jnp.int32),
        jax.ShapeDtypeStruct((E_PAD,), jnp.int32),
    ],
    mesh=_MESH,
    scratch_types=[
        pltpu.VMEM((N_PAD,), jnp.float32),
        pltpu.VMEM((DEG_CHUNK,), jnp.int32),
    ],
    compiler_params=_SC_PARAMS,
)
def _degree_kernel(src_ref, dst_ref, out_ref, srcp_ref, dstp_ref, cnt, idxbuf):
    c = lax.axis_index("c")
    s = lax.axis_index("s")
    z = jnp.zeros((16,), jnp.float32)
    ones = jnp.ones((16,), jnp.float32)

    @pl.loop(0, N_PAD // 16)
    def _zero(i):
        cnt[pl.ds(i * 16, 16)] = z

    def count(idx_ref, pad_ref, padval):
        @pl.loop(0, EDGES_PER_TILE // DEG_CHUNK)
        def _outer(o):
            base = pl.multiple_of(s * EDGES_PER_TILE + o * DEG_CHUNK, 8)
            pltpu.sync_copy(idx_ref.at[pl.ds(base, DEG_CHUNK)], idxbuf)
            pltpu.sync_copy(idxbuf, pad_ref.at[pl.ds(base, DEG_CHUNK)])

            @pl.loop(0, DEG_CHUNK // 16)
            def _inner(i):
                idxv = idxbuf[pl.ds(i * 16, 16)]
                plsc.addupdate_scatter(cnt, [idxv], ones)

        @pl.loop(0, PAD_PER_TILE // 16)
        def _fill(i):
            idxbuf[pl.ds(i * 16, 16)] = padval

        pbase = pl.multiple_of(E + s * PAD_PER_TILE, 8)
        pltpu.sync_copy(idxbuf.at[pl.ds(0, PAD_PER_TILE)],
                        pad_ref.at[pl.ds(pbase, PAD_PER_TILE)])

    @pl.when(c == 0)
    def _():
        count(src_ref, srcp_ref, jnp.zeros((16,), jnp.int32))

    @pl.when(c == 1)
    def _():
        count(dst_ref, dstp_ref, jnp.full((16,), N_PAD - 1, jnp.int32))

    obase = pl.multiple_of((c * 16 + s) * N_PAD, 128)
    pltpu.sync_copy(cnt, out_ref.at[pl.ds(obase, N_PAD)])


def _make_agg(n_chunks):
    @functools.partial(
        pl.kernel,
        out_type=jax.ShapeDtypeStruct((n_chunks, N_PAD, 32), jnp.float32),
        mesh=_MESH,
        scratch_types=[
            pltpu.VMEM_SHARED((N_PAD, 32), jnp.float32),
            pltpu.VMEM((4, S, K), jnp.int32),
            pltpu.VMEM((4, S, K), jnp.int32),
            pltpu.VMEM((3, S, K, 32), jnp.float32),
            pltpu.VMEM((ZROWS, 32), jnp.float32),
            pltpu.SemaphoreType.DMA,
            pltpu.SemaphoreType.DMA,
            pltpu.SemaphoreType.DMA,
            pltpu.SemaphoreType.DMA,
        ],
        compiler_params=_SC_PARAMS,
    )
    def _agg(src_ref, dst_ref, h_all, out_ref, acc, srcbuf, dstbuf, msg,
             zbuf, gsem_e, gsem_o, ssem, isem):
        c = lax.axis_index("c")
        s = lax.axis_index("s")
        z = jnp.zeros((16,), jnp.float32)
        zi = jnp.zeros((16,), jnp.int32)

        @pl.loop(0, ZROWS)
        def _zfill(i):
            zbuf[i, pl.ds(0, 16)] = z
            zbuf[i, pl.ds(16, 16)] = z

        for j in range(S):
            @pl.loop(0, K // 16)
            def _dzero(i, j=j):
                dstbuf[0, j, pl.ds(i * 16, 16)] = zi

        def do_chunk(h_ref, chunk):
            rbase = pl.multiple_of(s * ROWS_PER_TILE, 8)

            @pl.loop(0, ROWS_PER_TILE // ZROWS)
            def _za(j):
                pltpu.async_copy(zbuf, acc.at[pl.ds(rbase + j * ZROWS, ZROWS)],
                                 isem)

            @pl.loop(0, ROWS_PER_TILE // ZROWS)
            def _zw(j):
                pltpu.make_async_copy(
                    zbuf, acc.at[pl.ds(rbase, ZROWS)], isem).wait()
            plsc.subcore_barrier()

            for j in range(S):
                pltpu.async_copy(zbuf.at[pl.ds(0, K)],
                                 acc.at[dstbuf.at[0, j]], ssem, add=True)
            row0 = s * TILE_ROWS
            pltpu.sync_copy(src_ref.at[pl.ds(row0, S)], srcbuf.at[0])
            pltpu.sync_copy(dst_ref.at[pl.ds(row0, S)], dstbuf.at[0])
            pltpu.sync_copy(src_ref.at[pl.ds(row0 + S, S)], srcbuf.at[1])
            pltpu.sync_copy(dst_ref.at[pl.ds(row0 + S, S)], dstbuf.at[1])
            for j in range(S):
                pltpu.async_copy(h_ref.at[srcbuf.at[0, j]], msg.at[0, j],
                                 gsem_e)

            @pl.loop(0, OUTER)
            def _edges(o):
                ib = lax.rem(o, 4)
                ib1 = lax.rem(o + 1, 4)
                ib2 = lax.rem(o + 2, 4)
                mb = lax.rem(o, 3)
                mb1 = lax.rem(o + 1, 3)

                @pl.when(jnp.logical_and(o > 0, o < OUTER - 1))
                def _():
                    pltpu.make_async_copy(
                        src_ref.at[pl.ds(row0, S)], srcbuf.at[0], isem).wait()
                    pltpu.make_async_copy(
                        dst_ref.at[pl.ds(row0, S)], dstbuf.at[0], isem).wait()

                def fire_next(sem):
                    @pl.when(o < OUTER - 1)
                    def _():
                        for j in range(S):
                            pltpu.async_copy(h_ref.at[srcbuf.at[ib1, j]],
                                             msg.at[mb1, j], sem)

                def drain_cur(sem):
                    for j in range(S):
                        pltpu.make_async_copy(h_ref.at[srcbuf.at[0, j]],
                                              msg.at[0, j], sem).wait()

                @pl.when(lax.rem(o, 2) == 0)
                def _():
                    fire_next(gsem_o)
                    drain_cur(gsem_e)

                @pl.when(lax.rem(o, 2) == 1)
                def _():
                    fire_next(gsem_e)
                    drain_cur(gsem_o)

                @pl.when(o < OUTER - 2)
                def _():
                    rown = s * TILE_ROWS + (o + 2) * S
                    pltpu.async_copy(src_ref.at[pl.ds(rown, S)],
                                     srcbuf.at[ib2], isem)
                    pltpu.async_copy(dst_ref.at[pl.ds(rown, S)],
                                     dstbuf.at[ib2], isem)

                for j in range(S):
                    pltpu.make_async_copy(
                        msg.at[0, j], acc.at[dstbuf.at[0, j]], ssem).wait()
                for j in range(S):
                    pltpu.async_copy(msg.at[mb, j], acc.at[dstbuf.at[ib, j]],
                                     ssem, add=True)

            for j in range(S):
                pltpu.make_async_copy(
                    msg.at[0, j], acc.at[dstbuf.at[0, j]], ssem).wait()

            plsc.subcore_barrier()
            pltpu.sync_copy(
                acc.at[pl.ds(rbase, ROWS_PER_TILE)],
                out_ref.at[chunk, pl.ds(rbase, ROWS_PER_TILE)])

        for phase in range(n_chunks // 2):
            for core in range(2):
                chunk = 2 * phase + core

                @pl.when(c == core)
                def _(chunk=chunk):
                    do_chunk(h_all.at[chunk], chunk)

    return _agg


_agg4 = _make_agg(4)
_agg2 = _make_agg(2)


BLK = 1024
GRID = N_PAD // BLK


def _isqrts(pblk):
    deg = jnp.maximum(jnp.sum(pblk, axis=1), 1.0)
    return lax.rsqrt(deg)


def _split_store(out_ref, m):
    for q in range(out_ref.shape[0]):
        out_ref[q] = m[:, q * 32:(q + 1) * 32]


def _cat_chunks(a_ref):
    return jnp.concatenate([a_ref[q] for q in range(a_ref.shape[0])], axis=1)


def _stage1_body(feat_ref, p_ref, w1_ref, h1_ref):
    sc = _isqrts(p_ref[...])
    x = feat_ref[...] * sc[0][:, None]
    _split_store(h1_ref,
                 jnp.dot(x, w1_ref[...], preferred_element_type=jnp.float32))


_stage1 = pl.pallas_call(
    _stage1_body,
    grid=(GRID,),
    in_specs=[
        pl.BlockSpec((BLK, 128), lambda i: (i, 0)),
        pl.BlockSpec((2, 16, BLK), lambda i: (0, 0, i)),
        pl.BlockSpec((128, 128), lambda i: (0, 0)),
    ],
    out_specs=pl.BlockSpec((4, BLK, 32), lambda i: (0, i, 0)),
    out_shape=jax.ShapeDtypeStruct((4, N_PAD, 32), jnp.float32),
)


def _stage2_body(agg_ref, p_ref, b1_ref, w2_ref, h2_ref):
    sc = _isqrts(p_ref[...])
    x1 = jnp.maximum(_cat_chunks(agg_ref) * sc[1][:, None] + b1_ref[...], 0.0)
    _split_store(h2_ref,
                 jnp.dot(x1 * sc[0][:, None], w2_ref[...],
                         preferred_element_type=jnp.float32))


_stage2 = pl.pallas_call(
    _stage2_body,
    grid=(GRID,),
    in_specs=[
        pl.BlockSpec((4, BLK, 32), lambda i: (0, i, 0)),
        pl.BlockSpec((2, 16, BLK), lambda i: (0, 0, i)),
        pl.BlockSpec((1, 128), lambda i: (0, 0)),
        pl.BlockSpec((128, 64), lambda i: (0, 0)),
    ],
    out_specs=pl.BlockSpec((2, BLK, 32), lambda i: (0, i, 0)),
    out_shape=jax.ShapeDtypeStruct((2, N_PAD, 32), jnp.float32),
)


def _stage3_body(agg_ref, p_ref, b2_ref, wo1_ref, bo1_ref, wo2_ref, bo2_ref,
                 hid_ref, log_ref):
    sc = _isqrts(p_ref[...])
    hid = _cat_chunks(agg_ref) * sc[1][:, None] + b2_ref[...]
    hid_ref[...] = hid
    t = jnp.maximum(
        jnp.dot(hid, wo1_ref[...], preferred_element_type=jnp.float32)
        + bo1_ref[...], 0.0)
    log_ref[...] = jnp.dot(t, wo2_ref[...],
                           preferred_element_type=jnp.float32) + bo2_ref[...]


_stage3 = pl.pallas_call(
    _stage3_body,
    grid=(GRID,),
    in_specs=[
        pl.BlockSpec((2, BLK, 32), lambda i: (0, i, 0)),
        pl.BlockSpec((2, 16, BLK), lambda i: (0, 0, i)),
        pl.BlockSpec((1, 64), lambda i: (0, 0)),
        pl.BlockSpec((64, 128), lambda i: (0, 0)),
        pl.BlockSpec((1, 128), lambda i: (0, 0)),
        pl.BlockSpec((128, 128), lambda i: (0, 0)),
        pl.BlockSpec((1, 128), lambda i: (0, 0)),
    ],
    out_specs=[
        pl.BlockSpec((BLK, 64), lambda i: (i, 0)),
        pl.BlockSpec((BLK, 128), lambda i: (i, 0)),
    ],
    out_shape=[
        jax.ShapeDtypeStruct((N, 64), jnp.float32),
        jax.ShapeDtypeStruct((N, 128), jnp.float32),
    ],
)


def kernel(features, edge_index, W1, b1, W2, b2, Wo1, bo1, Wo2, bo2):
    n = features.shape[0]
    feat = features.reshape(n, -1)
    src = edge_index[0]
    dst = edge_index[1]

    partials_flat, srcp, dstp = _degree_kernel(src, dst)
    partials = partials_flat.reshape(2, 16, N_PAD)
    src2d = srcp.reshape(-1, K)
    dst2d = dstp.reshape(-1, K)

    h1t = _stage1(feat, partials, W1)
    agg1t = _agg4(src2d, dst2d, h1t)
    h2t = _stage2(agg1t, partials, b1.reshape(1, -1), W2)
    agg2t = _agg2(src2d, dst2d, h2t)

    wo2p = jnp.pad(Wo2, ((0, 0), (0, 128 - Wo2.shape[1])))
    bo2p = jnp.pad(bo2, (0, 128 - bo2.shape[0])).reshape(1, -1)
    hidden, log_p = _stage3(agg2t, partials, b2.reshape(1, -1), Wo1,
                            bo1.reshape(1, -1), wo2p, bo2p)
    logits = log_p[:, :2]
    return (logits, logits, hidden)
